# Initial kernel scaffold; baseline (speedup 1.0000x reference)
import jax
import jax.numpy as jnp
from jax.experimental import pallas as pl


def kernel(p, conv_pos_wf, conv_pos_wd, conv1_wf, conv1_wd, conv2_wf, conv2_wd, conv3_wf, conv3_wd):
    raise NotImplementedError("write your pallas kernel here")


# speedup vs baseline: 1.0131x; 1.0131x over previous
import functools

import jax
import jax.numpy as jnp
import numpy as np
from jax.experimental import pallas as pl
from jax.experimental.pallas import tpu as pltpu

EPS = 1e-6
NEG_SLOPE = 0.0
K_NBRS = 20
HID = 128


def _ceil_to(x, m):
    return ((x + m - 1) // m) * m


def _vn_lrelu(p, d, eps=EPS):
    dot = jnp.sum(p * d, axis=0, keepdims=True)
    dnsq = jnp.sum(d * d, axis=0, keepdims=True)
    neg = (dot < 0.0).astype(jnp.float32)
    corr = dot * pl.reciprocal(dnsq + eps, approx=True)
    return p - neg * corr * d


def _convpos_pool_kernel(x_ref, w_ref, pool_ref, o_ref, *, cout):
    tgk = x_ref.shape[1]
    cin = x_ref.shape[2]
    x2d = x_ref[...].reshape(3 * tgk, cin)
    y = jnp.dot(x2d, w_ref[...], preferred_element_type=jnp.float32)
    y = y.reshape(3, tgk, 2 * cout)
    out = _vn_lrelu(y[..., :cout], y[..., cout:])
    pool = pool_ref[...]
    for v in range(3):
        o_ref[v] = jnp.dot(pool, out[v],
                           preferred_element_type=jnp.float32).astype(o_ref.dtype)


def _conv_chain_kernel(x_ref, w1_ref, w2n_ref, w2p_ref, w3n_ref, w3p_ref,
                       o_ref, *, h, n_pts):
    bb = x_ref.shape[1]
    cin = x_ref.shape[3]
    inv_n = 1.0 / n_pts

    x2d = x_ref[...].reshape(3 * bb * n_pts, cin)
    y = jnp.dot(x2d, w1_ref[...], preferred_element_type=jnp.float32)
    y = y.reshape(3, bb * n_pts, 2 * h)
    net = _vn_lrelu(y[..., :h], y[..., h:])

    for wn_ref, wp_ref in ((w2n_ref, w2p_ref), (w3n_ref, w3p_ref)):
        pooled = jnp.sum(net.reshape(3, bb, n_pts, h), axis=2) * inv_n
        pproj = jnp.dot(pooled.reshape(3 * bb, h), wp_ref[...],
                        preferred_element_type=jnp.float32)
        y = jnp.dot(net.reshape(3 * bb * n_pts, h), wn_ref[...],
                    preferred_element_type=jnp.float32)
        y = y.reshape(3, bb, n_pts, 2 * h) + pproj.reshape(3, bb, 1, 2 * h)
        y = y.reshape(3, bb * n_pts, 2 * h)
        net = _vn_lrelu(y[..., :h], y[..., h:])

    pooled = jnp.sum(net.reshape(3, bb, n_pts, h), axis=2, keepdims=True) * inv_n
    o_ref[...] = jnp.broadcast_to(pooled, o_ref.shape).astype(o_ref.dtype)


def _vnt_normalize(w):
    return w / jnp.sum(w, axis=1, keepdims=True)


def _concat_w(wf, wd):
    return jnp.concatenate([_vnt_normalize(wf).T, _vnt_normalize(wd).T], axis=1)


def _graph_feat(p, k):
    B, N, _ = p.shape
    x = jnp.transpose(p, (0, 2, 1))
    inner = jnp.einsum("bdn,bdm->bnm", x, x)
    xx = jnp.sum(x * x, axis=1)
    pdist = 2.0 * inner - xx[:, :, None] - xx[:, None, :]
    idx = jax.lax.top_k(pdist, k)[1]
    nbrs = jax.vmap(lambda pts, id_: pts[id_])(p, idx)
    ctr = jnp.broadcast_to(p[:, :, None, :], nbrs.shape)
    cross = jnp.cross(nbrs, ctr, axis=-1)
    feat = jnp.stack([nbrs - ctr, ctr, cross], axis=3)
    feat = jnp.transpose(feat, (4, 0, 1, 2, 3))
    return feat.reshape(3, B * N * k, 3)


def _convpos_pool(feat, w, k):
    _, mk, cin = feat.shape
    g = mk // k
    cout = w.shape[1] // 2
    tg = 48
    gp = _ceil_to(g, tg)
    if gp != g:
        feat = feat.reshape(3, g, k, cin)
        feat = jnp.pad(feat, ((0, 0), (0, gp - g), (0, 0), (0, 0)))
        feat = feat.reshape(3, gp * k, cin)
    pool_mat = jnp.asarray(
        np.kron(np.eye(tg, dtype=np.float32), np.ones((1, k), np.float32)) / k)

    out = pl.pallas_call(
        functools.partial(_convpos_pool_kernel, cout=cout),
        out_shape=jax.ShapeDtypeStruct((3, gp, cout), jnp.float32),
        grid_spec=pltpu.PrefetchScalarGridSpec(
            num_scalar_prefetch=0,
            grid=(gp // tg,),
            in_specs=[
                pl.BlockSpec((3, tg * k, cin), lambda i: (0, i, 0)),
                pl.BlockSpec((cin, 2 * cout), lambda i: (0, 0)),
                pl.BlockSpec((tg, tg * k), lambda i: (0, 0)),
            ],
            out_specs=pl.BlockSpec((3, tg, cout), lambda i: (0, i, 0)),
        ),
        compiler_params=pltpu.CompilerParams(dimension_semantics=("parallel",)),
    )(feat, w, pool_mat)
    return out[:, :g, :]


def _conv_chain(net4, w1, w2, w3):
    _, B, N, _ = net4.shape
    h = HID
    w2n, w2p = w2[:h], w2[h:]
    w3n, w3p = w3[:h], w3[h:]
    bb = 8

    out = pl.pallas_call(
        functools.partial(_conv_chain_kernel, h=h, n_pts=N),
        out_shape=jax.ShapeDtypeStruct((3, B, 8, h), jnp.float32),
        grid_spec=pltpu.PrefetchScalarGridSpec(
            num_scalar_prefetch=0,
            grid=(B // bb,),
            in_specs=[
                pl.BlockSpec((3, bb, N, 64), lambda i: (0, i, 0, 0)),
                pl.BlockSpec((64, 2 * h), lambda i: (0, 0)),
                pl.BlockSpec((h, 2 * h), lambda i: (0, 0)),
                pl.BlockSpec((h, 2 * h), lambda i: (0, 0)),
                pl.BlockSpec((h, 2 * h), lambda i: (0, 0)),
                pl.BlockSpec((h, 2 * h), lambda i: (0, 0)),
            ],
            out_specs=pl.BlockSpec((3, bb, 8, h), lambda i: (0, i, 0, 0)),
        ),
        compiler_params=pltpu.CompilerParams(dimension_semantics=("parallel",)),
    )(net4, w1, w2n, w2p, w3n, w3p)
    return out[:, :, 0, :]


def kernel(p, conv_pos_wf, conv_pos_wd, conv1_wf, conv1_wd,
           conv2_wf, conv2_wd, conv3_wf, conv3_wd):
    B, N, _ = p.shape
    feat = _graph_feat(p, K_NBRS)
    net = _convpos_pool(feat, _concat_w(conv_pos_wf, conv_pos_wd), K_NBRS)
    net4 = net.reshape(3, B, N, 64)
    out = _conv_chain(net4, _concat_w(conv1_wf, conv1_wd),
                      _concat_w(conv2_wf, conv2_wd),
                      _concat_w(conv3_wf, conv3_wd))
    return jnp.transpose(out, (1, 2, 0))


# speedup vs baseline: 38.3132x; 37.8194x over previous
import functools

import jax
import jax.numpy as jnp
from jax.experimental import pallas as pl
from jax.experimental.pallas import tpu as pltpu

EPS = 1e-6
K_NBRS = 20
HID = 128
NEG_BIG = -3.0e38


def _knn_convpos_kernel(p_ref, pt_ref, w9t_ref, o0_ref, o1_ref, o2_ref,
                        *, n, k):
    p2d = p_ref[0]
    pT = pt_ref[0]

    inner = jnp.dot(p2d.astype(jnp.bfloat16), pT.astype(jnp.bfloat16),
                    preferred_element_type=jnp.float32)
    xx = pT * pT
    xxs = (xx[0:1, :] + xx[1:2, :]) + xx[2:3, :]
    xxc = jnp.sum(p2d * p2d, axis=1, keepdims=True)
    s = 2.0 * inner - xxc - xxs

    ctr0 = pT[0:1, :]
    ctr1 = pT[1:2, :]
    ctr2 = pT[2:3, :]

    acc = jnp.zeros((192, n), jnp.float32)
    for _ in range(k):
        mx = jnp.max(s, axis=0, keepdims=True)
        eq = s == mx
        oh = eq.astype(jnp.float32)
        s = jnp.where(eq, NEG_BIG, s)
        nbr = jnp.dot(pT, oh, preferred_element_type=jnp.float32)
        nb0, nb1, nb2 = nbr[0:1, :], nbr[1:2, :], nbr[2:3, :]
        x = jnp.concatenate([
            nb0 - ctr0, ctr0, nb1 * ctr2 - nb2 * ctr1,
            nb1 - ctr1, ctr1, nb2 * ctr0 - nb0 * ctr2,
            nb2 - ctr2, ctr2, nb0 * ctr1 - nb1 * ctr0,
        ], axis=0)
        y = jnp.dot(w9t_ref[...], x, preferred_element_type=jnp.float32)
        p0, d0 = y[0:64], y[64:128]
        p1, d1 = y[128:192], y[192:256]
        p2, d2 = y[256:320], y[320:384]
        dot = p0 * d0 + p1 * d1 + p2 * d2
        dnsq = d0 * d0 + d1 * d1 + d2 * d2
        sc = jnp.where(dot < 0.0,
                       dot * pl.reciprocal(dnsq + EPS, approx=True), 0.0)
        acc = acc + jnp.concatenate(
            [p0 - sc * d0, p1 - sc * d1, p2 - sc * d2], axis=0)

    out = acc * (1.0 / k)
    o0_ref[0] = out[0:64].T
    o1_ref[0] = out[64:128].T
    o2_ref[0] = out[128:192].T


def _knn_convpos(p, w9t, k):
    B, N, _ = p.shape
    pt = jnp.transpose(p, (0, 2, 1))
    out_sh = jax.ShapeDtypeStruct((B, N, 64), jnp.float32)
    return pl.pallas_call(
        functools.partial(_knn_convpos_kernel, n=N, k=k),
        out_shape=(out_sh, out_sh, out_sh),
        grid_spec=pltpu.PrefetchScalarGridSpec(
            num_scalar_prefetch=0,
            grid=(B,),
            in_specs=[
                pl.BlockSpec((1, N, 3), lambda i: (i, 0, 0)),
                pl.BlockSpec((1, 3, N), lambda i: (i, 0, 0)),
                pl.BlockSpec((384, 9), lambda i: (0, 0)),
            ],
            out_specs=(
                pl.BlockSpec((1, N, 64), lambda i: (i, 0, 0)),
                pl.BlockSpec((1, N, 64), lambda i: (i, 0, 0)),
                pl.BlockSpec((1, N, 64), lambda i: (i, 0, 0)),
            ),
        ),
        compiler_params=pltpu.CompilerParams(dimension_semantics=("parallel",)),
    )(p, pt, w9t)


def _vn_lrelu(p, d):
    dot = jnp.sum(p * d, axis=0, keepdims=True)
    dnsq = jnp.sum(d * d, axis=0, keepdims=True)
    neg = (dot < 0.0).astype(jnp.float32)
    corr = dot * pl.reciprocal(dnsq + EPS, approx=True)
    return p - neg * corr * d


def _conv_chain_kernel(x0_ref, x1_ref, x2_ref, w1_ref, w2n_ref, w2p_ref,
                       w3n_ref, w3p_ref, o_ref, *, h, n_pts):
    bb = x0_ref.shape[0]
    inv_n = 1.0 / n_pts
    x = jnp.stack([x0_ref[...], x1_ref[...], x2_ref[...]], axis=0)
    x2d = x.reshape(3 * bb * n_pts, x.shape[-1])
    y = jnp.dot(x2d, w1_ref[...], preferred_element_type=jnp.float32)
    y = y.reshape(3, bb * n_pts, 2 * h)
    net = _vn_lrelu(y[..., :h], y[..., h:])

    for wn_ref, wp_ref in ((w2n_ref, w2p_ref), (w3n_ref, w3p_ref)):
        pooled = jnp.sum(net.reshape(3, bb, n_pts, h), axis=2) * inv_n
        pproj = jnp.dot(pooled.reshape(3 * bb, h), wp_ref[...],
                        preferred_element_type=jnp.float32)
        y = jnp.dot(net.reshape(3 * bb * n_pts, h), wn_ref[...],
                    preferred_element_type=jnp.float32)
        y = y.reshape(3, bb, n_pts, 2 * h) + pproj.reshape(3, bb, 1, 2 * h)
        y = y.reshape(3, bb * n_pts, 2 * h)
        net = _vn_lrelu(y[..., :h], y[..., h:])

    pooled = jnp.sum(net.reshape(3, bb, n_pts, h), axis=2, keepdims=True) * inv_n
    o_ref[...] = jnp.broadcast_to(pooled, o_ref.shape)


def _conv_chain(net0, net1, net2, w1, w2, w3):
    B, N, _ = net0.shape
    h = HID
    w2n, w2p = w2[:h], w2[h:]
    w3n, w3p = w3[:h], w3[h:]
    bb = 8

    out = pl.pallas_call(
        functools.partial(_conv_chain_kernel, h=h, n_pts=N),
        out_shape=jax.ShapeDtypeStruct((3, B, 8, h), jnp.float32),
        grid_spec=pltpu.PrefetchScalarGridSpec(
            num_scalar_prefetch=0,
            grid=(B // bb,),
            in_specs=[
                pl.BlockSpec((bb, N, 64), lambda i: (i, 0, 0)),
                pl.BlockSpec((bb, N, 64), lambda i: (i, 0, 0)),
                pl.BlockSpec((bb, N, 64), lambda i: (i, 0, 0)),
                pl.BlockSpec((64, 2 * h), lambda i: (0, 0)),
                pl.BlockSpec((h, 2 * h), lambda i: (0, 0)),
                pl.BlockSpec((h, 2 * h), lambda i: (0, 0)),
                pl.BlockSpec((h, 2 * h), lambda i: (0, 0)),
                pl.BlockSpec((h, 2 * h), lambda i: (0, 0)),
            ],
            out_specs=pl.BlockSpec((3, bb, 8, h), lambda i: (0, i, 0, 0)),
        ),
        compiler_params=pltpu.CompilerParams(dimension_semantics=("parallel",)),
    )(net0, net1, net2, w1, w2n, w2p, w3n, w3p)
    return out[:, :, 0, :]


def _vnt_normalize(w):
    return w / jnp.sum(w, axis=1, keepdims=True)


def _concat_w(wf, wd):
    return jnp.concatenate([_vnt_normalize(wf).T, _vnt_normalize(wd).T], axis=1)


def _build_w9t(wf, wd):
    w = _concat_w(wf, wd)
    blocks = []
    for v in range(3):
        row = []
        for vp in range(3):
            row.append(w.T if v == vp else jnp.zeros((128, 3), jnp.float32))
        blocks.append(jnp.concatenate(row, axis=1))
    return jnp.concatenate(blocks, axis=0)


def kernel(p, conv_pos_wf, conv_pos_wd, conv1_wf, conv1_wd,
           conv2_wf, conv2_wd, conv3_wf, conv3_wd):
    w9t = _build_w9t(conv_pos_wf, conv_pos_wd)
    net0, net1, net2 = _knn_convpos(p, w9t, K_NBRS)
    out = _conv_chain(net0, net1, net2, _concat_w(conv1_wf, conv1_wd),
                      _concat_w(conv2_wf, conv2_wd),
                      _concat_w(conv3_wf, conv3_wd))
    return jnp.transpose(out, (1, 2, 0))


# speedup vs baseline: 72.5939x; 1.8947x over previous
import functools

import jax
import jax.numpy as jnp
import numpy as np
from jax.experimental import pallas as pl
from jax.experimental.pallas import tpu as pltpu

EPS = 1e-6
K_NBRS = 20
HID = 128
NEG_BIG = -3.0e38


def _knn_convpos_kernel(p_ref, pt_ref, w9t_ref, o0_ref, o1_ref, o2_ref,
                        *, n, k):
    p2d = p_ref[0]
    pT = pt_ref[0]

    inner = jnp.dot(p2d.astype(jnp.bfloat16), pT.astype(jnp.bfloat16),
                    preferred_element_type=jnp.float32)
    xx = pT * pT
    xxs = (xx[0:1, :] + xx[1:2, :]) + xx[2:3, :]
    xxc = jnp.sum(p2d * p2d, axis=1, keepdims=True)
    s = 2.0 * inner - xxc - xxs

    ctr0 = pT[0:1, :]
    ctr1 = pT[1:2, :]
    ctr2 = pT[2:3, :]

    acc = jnp.zeros((192, n), jnp.float32)
    for _ in range(k):
        mx = jnp.max(s, axis=0, keepdims=True)
        eq = s == mx
        oh = eq.astype(jnp.float32)
        s = jnp.where(eq, NEG_BIG, s)
        nbr = jnp.dot(pT, oh, preferred_element_type=jnp.float32)
        nb0, nb1, nb2 = nbr[0:1, :], nbr[1:2, :], nbr[2:3, :]
        x = jnp.concatenate([
            nb0 - ctr0, ctr0, nb1 * ctr2 - nb2 * ctr1,
            nb1 - ctr1, ctr1, nb2 * ctr0 - nb0 * ctr2,
            nb2 - ctr2, ctr2, nb0 * ctr1 - nb1 * ctr0,
        ], axis=0)
        y = jnp.dot(w9t_ref[...], x, preferred_element_type=jnp.float32)
        p0, d0 = y[0:64], y[64:128]
        p1, d1 = y[128:192], y[192:256]
        p2, d2 = y[256:320], y[320:384]
        dot = p0 * d0 + p1 * d1 + p2 * d2
        dnsq = d0 * d0 + d1 * d1 + d2 * d2
        sc = jnp.where(dot < 0.0,
                       dot * pl.reciprocal(dnsq + EPS, approx=True), 0.0)
        acc = acc + jnp.concatenate(
            [p0 - sc * d0, p1 - sc * d1, p2 - sc * d2], axis=0)

    out = acc * (1.0 / k)
    o0_ref[0] = out[0:64].T
    o1_ref[0] = out[64:128].T
    o2_ref[0] = out[128:192].T


def _knn_convpos(p, w9t, k):
    B, N, _ = p.shape
    pt = jnp.transpose(p, (0, 2, 1))
    out_sh = jax.ShapeDtypeStruct((B, N, 64), jnp.float32)
    return pl.pallas_call(
        functools.partial(_knn_convpos_kernel, n=N, k=k),
        out_shape=(out_sh, out_sh, out_sh),
        grid_spec=pltpu.PrefetchScalarGridSpec(
            num_scalar_prefetch=0,
            grid=(B,),
            in_specs=[
                pl.BlockSpec((1, N, 3), lambda i: (i, 0, 0)),
                pl.BlockSpec((1, 3, N), lambda i: (i, 0, 0)),
                pl.BlockSpec((384, 9), lambda i: (0, 0)),
            ],
            out_specs=(
                pl.BlockSpec((1, N, 64), lambda i: (i, 0, 0)),
                pl.BlockSpec((1, N, 64), lambda i: (i, 0, 0)),
                pl.BlockSpec((1, N, 64), lambda i: (i, 0, 0)),
            ),
        ),
        compiler_params=pltpu.CompilerParams(dimension_semantics=("parallel",)),
    )(p, pt, w9t)


def _vn_lrelu(p, d):
    dot = jnp.sum(p * d, axis=0, keepdims=True)
    dnsq = jnp.sum(d * d, axis=0, keepdims=True)
    neg = (dot < 0.0).astype(jnp.float32)
    corr = dot * pl.reciprocal(dnsq + EPS, approx=True)
    return p - neg * corr * d


def _conv_chain_kernel(x0_ref, x1_ref, x2_ref, w1_ref, w2n_ref, w2p_ref,
                       w3n_ref, w3p_ref, o_ref, *, h, n_pts):
    bb = x0_ref.shape[0]
    inv_n = 1.0 / n_pts
    x = jnp.stack([x0_ref[...], x1_ref[...], x2_ref[...]], axis=0)
    x2d = x.reshape(3 * bb * n_pts, x.shape[-1])
    y = jnp.dot(x2d, w1_ref[...], preferred_element_type=jnp.float32)
    y = y.reshape(3, bb * n_pts, 2 * h)
    net = _vn_lrelu(y[..., :h], y[..., h:])

    for wn_ref, wp_ref in ((w2n_ref, w2p_ref), (w3n_ref, w3p_ref)):
        pooled = jnp.sum(net.reshape(3, bb, n_pts, h), axis=2) * inv_n
        pproj = jnp.dot(pooled.reshape(3 * bb, h), wp_ref[...],
                        preferred_element_type=jnp.float32)
        y = jnp.dot(net.reshape(3 * bb * n_pts, h), wn_ref[...],
                    preferred_element_type=jnp.float32)
        y = y.reshape(3, bb, n_pts, 2 * h) + pproj.reshape(3, bb, 1, 2 * h)
        y = y.reshape(3, bb * n_pts, 2 * h)
        net = _vn_lrelu(y[..., :h], y[..., h:])

    pooled = jnp.sum(net.reshape(3, bb, n_pts, h), axis=2, keepdims=True) * inv_n
    o_ref[...] = jnp.broadcast_to(pooled, o_ref.shape)


def _conv_chain(net0, net1, net2, w1, w2, w3):
    B, N, _ = net0.shape
    h = HID
    w2n, w2p = w2[:h], w2[h:]
    w3n, w3p = w3[:h], w3[h:]
    bb = 8

    out = pl.pallas_call(
        functools.partial(_conv_chain_kernel, h=h, n_pts=N),
        out_shape=jax.ShapeDtypeStruct((3, B, 8, h), jnp.float32),
        grid_spec=pltpu.PrefetchScalarGridSpec(
            num_scalar_prefetch=0,
            grid=(B // bb,),
            in_specs=[
                pl.BlockSpec((bb, N, 64), lambda i: (i, 0, 0)),
                pl.BlockSpec((bb, N, 64), lambda i: (i, 0, 0)),
                pl.BlockSpec((bb, N, 64), lambda i: (i, 0, 0)),
                pl.BlockSpec((64, 2 * h), lambda i: (0, 0)),
                pl.BlockSpec((h, 2 * h), lambda i: (0, 0)),
                pl.BlockSpec((h, 2 * h), lambda i: (0, 0)),
                pl.BlockSpec((h, 2 * h), lambda i: (0, 0)),
                pl.BlockSpec((h, 2 * h), lambda i: (0, 0)),
            ],
            out_specs=pl.BlockSpec((3, bb, 8, h), lambda i: (0, i, 0, 0)),
        ),
        compiler_params=pltpu.CompilerParams(dimension_semantics=("parallel",)),
    )(net0, net1, net2, w1, w2n, w2p, w3n, w3p)
    return out[:, :, 0, :]


def _vnt_normalize(w):
    return w / jnp.sum(w, axis=1, keepdims=True)


def _concat_w(wf, wd):
    return jnp.concatenate([_vnt_normalize(wf).T, _vnt_normalize(wd).T], axis=1)


def _build_w9t(wf, wd):
    w = _concat_w(wf, wd)
    blocks = []
    for v in range(3):
        row = []
        for vp in range(3):
            row.append(w.T if v == vp else jnp.zeros((128, 3), jnp.float32))
        blocks.append(jnp.concatenate(row, axis=1))
    return jnp.concatenate(blocks, axis=0)


def _forward(p, w9t, w1, w2, w3):
    net0, net1, net2 = _knn_convpos(p, w9t, K_NBRS)
    out = _conv_chain(net0, net1, net2, w1, w2, w3)
    return jnp.transpose(out, (1, 2, 0))


def kernel(p, conv_pos_wf, conv_pos_wd, conv1_wf, conv1_wd,
           conv2_wf, conv2_wd, conv3_wf, conv3_wd):
    w9t = _build_w9t(conv_pos_wf, conv_pos_wd)
    w1 = _concat_w(conv1_wf, conv1_wd)
    w2 = _concat_w(conv2_wf, conv2_wd)
    w3 = _concat_w(conv3_wf, conv3_wd)
    devs = jax.devices()
    mesh = jax.sharding.Mesh(np.array(devs), ("b",))
    pspec = jax.sharding.PartitionSpec
    f = jax.shard_map(
        _forward, mesh=mesh,
        in_specs=(pspec("b"), pspec(None, None), pspec(None, None),
                  pspec(None, None), pspec(None, None)),
        out_specs=pspec("b"),
        check_vma=False,
    )
    return f(p, w9t, w1, w2, w3)


# speedup vs baseline: 74.4696x; 1.0258x over previous
import functools

import jax
import jax.numpy as jnp
import numpy as np
from jax.experimental import pallas as pl
from jax.experimental.pallas import tpu as pltpu

EPS = 1e-6
K_NBRS = 20
HID = 128
NEG_BIG = -3.0e38


def _knn_convpos_kernel(p_ref, pt_ref, w9t_ref, o0_ref, o1_ref, o2_ref,
                        *, n, k, bb):
    for bi in range(bb):
        p2d = p_ref[bi]
        pT = pt_ref[bi]

        inner = jnp.dot(p2d.astype(jnp.bfloat16), pT.astype(jnp.bfloat16),
                        preferred_element_type=jnp.float32)
        xx = pT * pT
        xxs = (xx[0:1, :] + xx[1:2, :]) + xx[2:3, :]
        xxc = jnp.sum(p2d * p2d, axis=1, keepdims=True)
        s = 2.0 * inner - xxc - xxs

        ctr0 = pT[0:1, :]
        ctr1 = pT[1:2, :]
        ctr2 = pT[2:3, :]

        acc = jnp.zeros((192, n), jnp.float32)
        for _ in range(k):
            mx = jnp.max(s, axis=0, keepdims=True)
            eq = s == mx
            oh = eq.astype(jnp.float32)
            s = jnp.where(eq, NEG_BIG, s)
            nbr = jnp.dot(pT, oh, preferred_element_type=jnp.float32)
            nb0, nb1, nb2 = nbr[0:1, :], nbr[1:2, :], nbr[2:3, :]
            x = jnp.concatenate([
                nb0 - ctr0, ctr0, nb1 * ctr2 - nb2 * ctr1,
                nb1 - ctr1, ctr1, nb2 * ctr0 - nb0 * ctr2,
                nb2 - ctr2, ctr2, nb0 * ctr1 - nb1 * ctr0,
            ], axis=0)
            y = jnp.dot(w9t_ref[...], x, preferred_element_type=jnp.float32)
            p0, d0 = y[0:64], y[64:128]
            p1, d1 = y[128:192], y[192:256]
            p2, d2 = y[256:320], y[320:384]
            dot = p0 * d0 + p1 * d1 + p2 * d2
            dnsq = d0 * d0 + d1 * d1 + d2 * d2
            sc = jnp.where(dot < 0.0,
                           dot * pl.reciprocal(dnsq + EPS, approx=True), 0.0)
            acc = acc + jnp.concatenate(
                [p0 - sc * d0, p1 - sc * d1, p2 - sc * d2], axis=0)

        out = acc * (1.0 / k)
        o0_ref[bi] = out[0:64].T
        o1_ref[bi] = out[64:128].T
        o2_ref[bi] = out[128:192].T


def _knn_convpos(p, w9t, k):
    B, N, _ = p.shape
    bb = 2
    pt = jnp.transpose(p, (0, 2, 1))
    out_sh = jax.ShapeDtypeStruct((B, N, 64), jnp.float32)
    return pl.pallas_call(
        functools.partial(_knn_convpos_kernel, n=N, k=k, bb=bb),
        out_shape=(out_sh, out_sh, out_sh),
        grid_spec=pltpu.PrefetchScalarGridSpec(
            num_scalar_prefetch=0,
            grid=(B // bb,),
            in_specs=[
                pl.BlockSpec((bb, N, 3), lambda i: (i, 0, 0)),
                pl.BlockSpec((bb, 3, N), lambda i: (i, 0, 0)),
                pl.BlockSpec((384, 9), lambda i: (0, 0)),
            ],
            out_specs=(
                pl.BlockSpec((bb, N, 64), lambda i: (i, 0, 0)),
                pl.BlockSpec((bb, N, 64), lambda i: (i, 0, 0)),
                pl.BlockSpec((bb, N, 64), lambda i: (i, 0, 0)),
            ),
        ),
        compiler_params=pltpu.CompilerParams(dimension_semantics=("parallel",)),
    )(p, pt, w9t)


def _vn_lrelu(p, d):
    dot = jnp.sum(p * d, axis=0, keepdims=True)
    dnsq = jnp.sum(d * d, axis=0, keepdims=True)
    neg = (dot < 0.0).astype(jnp.float32)
    corr = dot * pl.reciprocal(dnsq + EPS, approx=True)
    return p - neg * corr * d


def _conv_chain_kernel(x0_ref, x1_ref, x2_ref, w1_ref, w2n_ref, w2p_ref,
                       w3n_ref, w3p_ref, o_ref, *, h, n_pts):
    bb = x0_ref.shape[0]
    inv_n = 1.0 / n_pts
    x = jnp.stack([x0_ref[...], x1_ref[...], x2_ref[...]], axis=0)
    x2d = x.reshape(3 * bb * n_pts, x.shape[-1])
    y = jnp.dot(x2d, w1_ref[...], preferred_element_type=jnp.float32)
    y = y.reshape(3, bb * n_pts, 2 * h)
    net = _vn_lrelu(y[..., :h], y[..., h:])

    for wn_ref, wp_ref in ((w2n_ref, w2p_ref), (w3n_ref, w3p_ref)):
        pooled = jnp.sum(net.reshape(3, bb, n_pts, h), axis=2) * inv_n
        pproj = jnp.dot(pooled.reshape(3 * bb, h), wp_ref[...],
                        preferred_element_type=jnp.float32)
        y = jnp.dot(net.reshape(3 * bb * n_pts, h), wn_ref[...],
                    preferred_element_type=jnp.float32)
        y = y.reshape(3, bb, n_pts, 2 * h) + pproj.reshape(3, bb, 1, 2 * h)
        y = y.reshape(3, bb * n_pts, 2 * h)
        net = _vn_lrelu(y[..., :h], y[..., h:])

    pooled = jnp.sum(net.reshape(3, bb, n_pts, h), axis=2, keepdims=True) * inv_n
    o_ref[...] = jnp.broadcast_to(pooled, o_ref.shape)


def _conv_chain(net0, net1, net2, w1, w2, w3):
    B, N, _ = net0.shape
    h = HID
    w2n, w2p = w2[:h], w2[h:]
    w3n, w3p = w3[:h], w3[h:]
    bb = 8

    out = pl.pallas_call(
        functools.partial(_conv_chain_kernel, h=h, n_pts=N),
        out_shape=jax.ShapeDtypeStruct((3, B, 8, h), jnp.float32),
        grid_spec=pltpu.PrefetchScalarGridSpec(
            num_scalar_prefetch=0,
            grid=(B // bb,),
            in_specs=[
                pl.BlockSpec((bb, N, 64), lambda i: (i, 0, 0)),
                pl.BlockSpec((bb, N, 64), lambda i: (i, 0, 0)),
                pl.BlockSpec((bb, N, 64), lambda i: (i, 0, 0)),
                pl.BlockSpec((64, 2 * h), lambda i: (0, 0)),
                pl.BlockSpec((h, 2 * h), lambda i: (0, 0)),
                pl.BlockSpec((h, 2 * h), lambda i: (0, 0)),
                pl.BlockSpec((h, 2 * h), lambda i: (0, 0)),
                pl.BlockSpec((h, 2 * h), lambda i: (0, 0)),
            ],
            out_specs=pl.BlockSpec((3, bb, 8, h), lambda i: (0, i, 0, 0)),
        ),
        compiler_params=pltpu.CompilerParams(dimension_semantics=("parallel",)),
    )(net0, net1, net2, w1, w2n, w2p, w3n, w3p)
    return out[:, :, 0, :]


def _vnt_normalize(w):
    return w / jnp.sum(w, axis=1, keepdims=True)


def _concat_w(wf, wd):
    return jnp.concatenate([_vnt_normalize(wf).T, _vnt_normalize(wd).T], axis=1)


def _build_w9t(wf, wd):
    w = _concat_w(wf, wd)
    blocks = []
    for v in range(3):
        row = []
        for vp in range(3):
            row.append(w.T if v == vp else jnp.zeros((128, 3), jnp.float32))
        blocks.append(jnp.concatenate(row, axis=1))
    return jnp.concatenate(blocks, axis=0)


def _forward(p, w9t, w1, w2, w3):
    net0, net1, net2 = _knn_convpos(p, w9t, K_NBRS)
    out = _conv_chain(net0, net1, net2, w1, w2, w3)
    return jnp.transpose(out, (1, 2, 0))


def kernel(p, conv_pos_wf, conv_pos_wd, conv1_wf, conv1_wd,
           conv2_wf, conv2_wd, conv3_wf, conv3_wd):
    w9t = _build_w9t(conv_pos_wf, conv_pos_wd)
    w1 = _concat_w(conv1_wf, conv1_wd)
    w2 = _concat_w(conv2_wf, conv2_wd)
    w3 = _concat_w(conv3_wf, conv3_wd)
    am = jax.sharding.get_abstract_mesh()
    if am is not None and not am.empty:
        mesh, axis = am, am.axis_names[0]
    else:
        mesh, axis = jax.sharding.Mesh(np.array(jax.devices()), ("b",)), "b"
    pspec = jax.sharding.PartitionSpec
    f = jax.shard_map(
        _forward, mesh=mesh,
        in_specs=(pspec(axis), pspec(None, None), pspec(None, None),
                  pspec(None, None), pspec(None, None)),
        out_specs=pspec(axis),
        check_vma=False,
    )
    return f(p, w9t, w1, w2, w3)


# speedup vs baseline: 79.6013x; 1.0689x over previous
import functools

import jax
import jax.numpy as jnp
import numpy as np
from jax.experimental import pallas as pl
from jax.experimental.pallas import tpu as pltpu

EPS = 1e-6
K_NBRS = 20
HID = 128
NEG_BIG = -3.0e38


def _knn_convpos_kernel(p_ref, pt_ref, wd9t_ref, wp9t_ref, qt_ref,
                        o0_ref, o1_ref, o2_ref, *, n, k, bb):
    for bi in range(bb):
        p2d = p_ref[bi]
        pT = pt_ref[bi]

        inner = jnp.dot(p2d.astype(jnp.bfloat16), pT.astype(jnp.bfloat16),
                        preferred_element_type=jnp.float32)
        xx = pT * pT
        xxs = (xx[0:1, :] + xx[1:2, :]) + xx[2:3, :]
        xxc = jnp.sum(p2d * p2d, axis=1, keepdims=True)
        s = 2.0 * inner - xxc - xxs

        ctr0 = pT[0:1, :]
        ctr1 = pT[1:2, :]
        ctr2 = pT[2:3, :]

        acc = jnp.zeros((192, n), jnp.float32)
        xpool = jnp.zeros((9, n), jnp.float32)
        for _ in range(k):
            mx = jnp.max(s, axis=0, keepdims=True)
            eq = s == mx
            oh = eq.astype(jnp.float32)
            s = jnp.where(eq, NEG_BIG, s)
            nbr = jnp.dot(pT, oh, preferred_element_type=jnp.float32)
            nb0, nb1, nb2 = nbr[0:1, :], nbr[1:2, :], nbr[2:3, :]
            rows = [
                nb0 - ctr0, ctr0, nb1 * ctr2 - nb2 * ctr1,
                nb1 - ctr1, ctr1, nb2 * ctr0 - nb0 * ctr2,
                nb2 - ctr2, ctr2, nb0 * ctr1 - nb1 * ctr0,
            ]
            x = jnp.concatenate(rows, axis=0)
            g = jnp.concatenate([
                rows[0] * rows[0] + rows[3] * rows[3] + rows[6] * rows[6],
                rows[1] * rows[1] + rows[4] * rows[4] + rows[7] * rows[7],
                rows[2] * rows[2] + rows[5] * rows[5] + rows[8] * rows[8],
                rows[0] * rows[1] + rows[3] * rows[4] + rows[6] * rows[7],
                rows[0] * rows[2] + rows[3] * rows[5] + rows[6] * rows[8],
                rows[1] * rows[2] + rows[4] * rows[5] + rows[7] * rows[8],
            ], axis=0)
            dd = jnp.dot(qt_ref[...], g, preferred_element_type=jnp.float32)
            yd = jnp.dot(wd9t_ref[...], x, preferred_element_type=jnp.float32)
            dotv, dnsq = dd[0:64], dd[64:128]
            sc = jnp.where(dotv < 0.0,
                           dotv * pl.reciprocal(dnsq + EPS, approx=True), 0.0)
            acc = acc + jnp.concatenate(
                [sc * yd[0:64], sc * yd[64:128], sc * yd[128:192]], axis=0)
            xpool = xpool + x

        yp = jnp.dot(wp9t_ref[...], xpool, preferred_element_type=jnp.float32)
        out = (yp - acc) * (1.0 / k)
        o0_ref[bi] = out[0:64].T
        o1_ref[bi] = out[64:128].T
        o2_ref[bi] = out[128:192].T


def _knn_convpos(p, wd9t, wp9t, qt, k):
    B, N, _ = p.shape
    bb = 2
    pt = jnp.transpose(p, (0, 2, 1))
    out_sh = jax.ShapeDtypeStruct((B, N, 64), jnp.float32)
    return pl.pallas_call(
        functools.partial(_knn_convpos_kernel, n=N, k=k, bb=bb),
        out_shape=(out_sh, out_sh, out_sh),
        grid_spec=pltpu.PrefetchScalarGridSpec(
            num_scalar_prefetch=0,
            grid=(B // bb,),
            in_specs=[
                pl.BlockSpec((bb, N, 3), lambda i: (i, 0, 0)),
                pl.BlockSpec((bb, 3, N), lambda i: (i, 0, 0)),
                pl.BlockSpec((192, 9), lambda i: (0, 0)),
                pl.BlockSpec((192, 9), lambda i: (0, 0)),
                pl.BlockSpec((128, 6), lambda i: (0, 0)),
            ],
            out_specs=(
                pl.BlockSpec((bb, N, 64), lambda i: (i, 0, 0)),
                pl.BlockSpec((bb, N, 64), lambda i: (i, 0, 0)),
                pl.BlockSpec((bb, N, 64), lambda i: (i, 0, 0)),
            ),
        ),
        compiler_params=pltpu.CompilerParams(dimension_semantics=("parallel",)),
    )(p, pt, wd9t, wp9t, qt)


def _vn_lrelu(p, d):
    dot = jnp.sum(p * d, axis=0, keepdims=True)
    dnsq = jnp.sum(d * d, axis=0, keepdims=True)
    neg = (dot < 0.0).astype(jnp.float32)
    corr = dot * pl.reciprocal(dnsq + EPS, approx=True)
    return p - neg * corr * d


def _conv_chain_kernel(x0_ref, x1_ref, x2_ref, w1_ref, w2n_ref, w2p_ref,
                       w3n_ref, w3p_ref, o_ref, *, h, n_pts):
    bb = x0_ref.shape[0]
    inv_n = 1.0 / n_pts
    x = jnp.stack([x0_ref[...], x1_ref[...], x2_ref[...]], axis=0)
    x2d = x.reshape(3 * bb * n_pts, x.shape[-1])
    y = jnp.dot(x2d, w1_ref[...], preferred_element_type=jnp.float32)
    y = y.reshape(3, bb * n_pts, 2 * h)
    net = _vn_lrelu(y[..., :h], y[..., h:])

    for wn_ref, wp_ref in ((w2n_ref, w2p_ref), (w3n_ref, w3p_ref)):
        pooled = jnp.sum(net.reshape(3, bb, n_pts, h), axis=2) * inv_n
        pproj = jnp.dot(pooled.reshape(3 * bb, h), wp_ref[...],
                        preferred_element_type=jnp.float32)
        y = jnp.dot(net.reshape(3 * bb * n_pts, h), wn_ref[...],
                    preferred_element_type=jnp.float32)
        y = y.reshape(3, bb, n_pts, 2 * h) + pproj.reshape(3, bb, 1, 2 * h)
        y = y.reshape(3, bb * n_pts, 2 * h)
        net = _vn_lrelu(y[..., :h], y[..., h:])

    pooled = jnp.sum(net.reshape(3, bb, n_pts, h), axis=2, keepdims=True) * inv_n
    o_ref[...] = jnp.broadcast_to(pooled, o_ref.shape)


def _conv_chain(net0, net1, net2, w1, w2, w3):
    B, N, _ = net0.shape
    h = HID
    w2n, w2p = w2[:h], w2[h:]
    w3n, w3p = w3[:h], w3[h:]
    bb = 8

    out = pl.pallas_call(
        functools.partial(_conv_chain_kernel, h=h, n_pts=N),
        out_shape=jax.ShapeDtypeStruct((3, B, 8, h), jnp.float32),
        grid_spec=pltpu.PrefetchScalarGridSpec(
            num_scalar_prefetch=0,
            grid=(B // bb,),
            in_specs=[
                pl.BlockSpec((bb, N, 64), lambda i: (i, 0, 0)),
                pl.BlockSpec((bb, N, 64), lambda i: (i, 0, 0)),
                pl.BlockSpec((bb, N, 64), lambda i: (i, 0, 0)),
                pl.BlockSpec((64, 2 * h), lambda i: (0, 0)),
                pl.BlockSpec((h, 2 * h), lambda i: (0, 0)),
                pl.BlockSpec((h, 2 * h), lambda i: (0, 0)),
                pl.BlockSpec((h, 2 * h), lambda i: (0, 0)),
                pl.BlockSpec((h, 2 * h), lambda i: (0, 0)),
            ],
            out_specs=pl.BlockSpec((3, bb, 8, h), lambda i: (0, i, 0, 0)),
        ),
        compiler_params=pltpu.CompilerParams(dimension_semantics=("parallel",)),
    )(net0, net1, net2, w1, w2n, w2p, w3n, w3p)
    return out[:, :, 0, :]


def _vnt_normalize(w):
    return w / jnp.sum(w, axis=1, keepdims=True)


def _concat_w(wf, wd):
    return jnp.concatenate([_vnt_normalize(wf).T, _vnt_normalize(wd).T], axis=1)


def _build_convpos_mats(wf, wd):
    wfn = _vnt_normalize(wf).T
    wdn = _vnt_normalize(wd).T

    def blockdiag(w3):
        z = jnp.zeros((64, 3), jnp.float32)
        rows = []
        for v in range(3):
            row = [w3.T if v == vp else z for vp in range(3)]
            rows.append(jnp.concatenate(row, axis=1))
        return jnp.concatenate(rows, axis=0)

    pairs = [(0, 0), (1, 1), (2, 2), (0, 1), (0, 2), (1, 2)]
    qdot, qdn = [], []
    for c, cp in pairs:
        if c == cp:
            qdot.append(wfn[c] * wdn[c])
            qdn.append(wdn[c] * wdn[c])
        else:
            qdot.append(wfn[c] * wdn[cp] + wfn[cp] * wdn[c])
            qdn.append(2.0 * wdn[c] * wdn[cp])
    qt = jnp.concatenate([jnp.stack(qdot, axis=1),
                          jnp.stack(qdn, axis=1)], axis=0)
    return blockdiag(wdn), blockdiag(wfn), qt


def _forward(p, wd9t, wp9t, qt, w1, w2, w3):
    net0, net1, net2 = _knn_convpos(p, wd9t, wp9t, qt, K_NBRS)
    out = _conv_chain(net0, net1, net2, w1, w2, w3)
    return jnp.transpose(out, (1, 2, 0))


def kernel(p, conv_pos_wf, conv_pos_wd, conv1_wf, conv1_wd,
           conv2_wf, conv2_wd, conv3_wf, conv3_wd):
    wd9t, wp9t, qt = _build_convpos_mats(conv_pos_wf, conv_pos_wd)
    w1 = _concat_w(conv1_wf, conv1_wd)
    w2 = _concat_w(conv2_wf, conv2_wd)
    w3 = _concat_w(conv3_wf, conv3_wd)
    am = jax.sharding.get_abstract_mesh()
    if am is not None and not am.empty:
        mesh, axis = am, am.axis_names[0]
    else:
        mesh, axis = jax.sharding.Mesh(np.array(jax.devices()), ("b",)), "b"
    pspec = jax.sharding.PartitionSpec
    wspec = pspec(None, None)
    f = jax.shard_map(
        _forward, mesh=mesh,
        in_specs=(pspec(axis), wspec, wspec, wspec, wspec, wspec, wspec),
        out_specs=pspec(axis),
        check_vma=False,
    )
    return f(p, wd9t, wp9t, qt, w1, w2, w3)
